# Initial kernel scaffold; baseline (speedup 1.0000x reference)
import functools

import jax
import jax.numpy as jnp
from jax import lax
from jax.experimental import pallas as pl
from jax.experimental.pallas import tpu as pltpu
from jax.experimental.pallas import tpu_sc as plsc


def _make_gather(n_total: int, vocab: int, d: int):
    info = plsc.get_sparse_core_info()
    nc, ns = info.num_cores, info.num_subcores
    nw = nc * ns

    assert n_total % nw == 0
    n_w = n_total // nw

    chunk = 1024
    while n_w % chunk:
        chunk //= 2
    n_chunks = n_w // chunk

    mesh = plsc.VectorSubcoreMesh(core_axis_name="c", subcore_axis_name="s")

    @functools.partial(
        pl.kernel,
        mesh=mesh,
        out_type=jax.ShapeDtypeStruct((n_total, d), jnp.float32),
        scratch_types=[
            pltpu.VMEM((chunk,), jnp.int32),
            pltpu.VMEM((chunk, d), jnp.float32),
            pltpu.SemaphoreType.DMA,
        ],
    )
    def gather_kernel(idx_hbm, table_hbm, out_hbm, idx_v, rows_v, sem):
        wid = lax.axis_index("s") * nc + lax.axis_index("c")
        base = wid * n_w

        def body(i, carry):
            off = base + i * chunk
            pltpu.sync_copy(idx_hbm.at[pl.ds(off, chunk)], idx_v)
            pltpu.async_copy(table_hbm.at[idx_v], rows_v, sem).wait()
            pltpu.sync_copy(rows_v, out_hbm.at[pl.ds(off, chunk)])
            return carry

        lax.fori_loop(0, n_chunks, body, 0)

    return gather_kernel


def kernel(indices, table):
    b, l = indices.shape
    vocab, d = table.shape
    n_total = b * l
    idx_flat = indices.reshape(n_total)
    out = _make_gather(n_total, vocab, d)(idx_flat, table)
    return out.reshape(b, l, d)


# speedup vs baseline: 1.4601x; 1.4601x over previous
import functools

import jax
import jax.numpy as jnp
from jax import lax
from jax.experimental import pallas as pl
from jax.experimental.pallas import tpu as pltpu
from jax.experimental.pallas import tpu_sc as plsc


def _make_gather(n_total: int, vocab: int, d: int):
    info = plsc.get_sparse_core_info()
    nc, ns = info.num_cores, info.num_subcores
    nw = nc * ns

    assert n_total % nw == 0
    n_w = n_total // nw

    chunk = 1024
    while n_w % chunk:
        chunk //= 2
    n_chunks = n_w // chunk

    mesh = plsc.VectorSubcoreMesh(core_axis_name="c", subcore_axis_name="s")

    @functools.partial(
        pl.kernel,
        mesh=mesh,
        compiler_params=pltpu.CompilerParams(use_tc_tiling_on_sc=False),
        out_type=jax.ShapeDtypeStruct((n_total, d), jnp.float32),
        scratch_types=[
            pltpu.VMEM((chunk,), jnp.int32),
            pltpu.VMEM((chunk, d), jnp.float32),
            pltpu.SemaphoreType.DMA,
        ],
    )
    def gather_kernel(idx_hbm, table_hbm, out_hbm, idx_v, rows_v, sem):
        wid = lax.axis_index("s") * nc + lax.axis_index("c")
        base = wid * n_w

        def body(i, carry):
            off = base + i * chunk
            pltpu.sync_copy(idx_hbm.at[pl.ds(off, chunk)], idx_v)
            pltpu.async_copy(table_hbm.at[idx_v], rows_v, sem).wait()
            pltpu.sync_copy(rows_v, out_hbm.at[pl.ds(off, chunk)])
            return carry

        lax.fori_loop(0, n_chunks, body, 0)

    return gather_kernel


def kernel(indices, table):
    b, l = indices.shape
    vocab, d = table.shape
    n_total = b * l
    idx_flat = indices.reshape(n_total)
    out = _make_gather(n_total, vocab, d)(idx_flat, table)
    return out.reshape(b, l, d)


# speedup vs baseline: 1.5013x; 1.0283x over previous
import functools

import jax
import jax.numpy as jnp
from jax import lax
from jax.experimental import pallas as pl
from jax.experimental.pallas import tpu as pltpu
from jax.experimental.pallas import tpu_sc as plsc

_NBUF = 4


def _make_gather(n_total: int, vocab: int, d: int):
    info = plsc.get_sparse_core_info()
    nc, ns = info.num_cores, info.num_subcores
    nw = nc * ns

    assert n_total % nw == 0
    n_w = n_total // nw

    chunk = 800
    while n_w % (chunk * _NBUF):
        chunk //= 2
    n_chunks = n_w // chunk
    n_groups = n_chunks // _NBUF

    mesh = plsc.VectorSubcoreMesh(core_axis_name="c", subcore_axis_name="s")

    @functools.partial(
        pl.kernel,
        mesh=mesh,
        compiler_params=pltpu.CompilerParams(use_tc_tiling_on_sc=False),
        out_type=jax.ShapeDtypeStruct((n_total, d), jnp.float32),
        scratch_types=[
            pltpu.VMEM((n_w,), jnp.int32),
        ]
        + [pltpu.VMEM((chunk, d), jnp.float32)] * _NBUF
        + [pltpu.SemaphoreType.DMA] * (2 * _NBUF),
    )
    def gather_kernel(idx_hbm, table_hbm, out_hbm, idx_v, *scratch):
        rows = scratch[:_NBUF]
        sem_g = scratch[_NBUF : 2 * _NBUF]
        sem_s = scratch[2 * _NBUF :]
        wid = lax.axis_index("s") * nc + lax.axis_index("c")
        base = wid * n_w

        pltpu.sync_copy(idx_hbm.at[pl.ds(base, n_w)], idx_v)

        def gather_copy(i, b):
            return pltpu.make_async_copy(
                table_hbm.at[idx_v.at[pl.ds(i * chunk, chunk)]], rows[b], sem_g[b]
            )

        def store_copy(i, b):
            return pltpu.make_async_copy(
                rows[b], out_hbm.at[pl.ds(base + i * chunk, chunk)], sem_s[b]
            )

        gather_copy(0, 0).start()
        gather_copy(1, 1).start()
        gather_copy(2, 2).start()
        gather_copy(0, 0).wait()
        store_copy(0, 0).start()
        gather_copy(3, 3).start()
        gather_copy(1, 1).wait()
        store_copy(1, 1).start()

        def body(g, carry):
            for b in range(_NBUF):
                i = g * _NBUF + b
                bp = (b + 2) % _NBUF
                store_copy(i - _NBUF, b).wait()
                gather_copy(i, b).start()
                gather_copy(i - 2, bp).wait()
                store_copy(i - 2, bp).start()
            return carry

        lax.fori_loop(1, n_groups, body, 0)

        last = n_chunks - 2
        for k in range(2):
            i = last + k
            b = i % _NBUF
            gather_copy(i, b).wait()
            store_copy(i, b).start()
        for k in range(_NBUF):
            i = n_chunks - _NBUF + k
            store_copy(i, i % _NBUF).wait()

    return gather_kernel


def kernel(indices, table):
    b, l = indices.shape
    vocab, d = table.shape
    n_total = b * l
    idx_flat = indices.reshape(n_total)
    out = _make_gather(n_total, vocab, d)(idx_flat, table)
    return out.reshape(b, l, d)
